# baseline (speedup 1.0000x reference)
import functools
import jax
import jax.numpy as jnp
from jax import lax
from jax.experimental import pallas as pl
from jax.experimental.pallas import tpu as pltpu
from jax.experimental.pallas import tpu_sc as plsc

B, N, H, D = 1024, 1000, 31, 128
B_SC = 768
B_TC = B - B_SC
NW = 32
NB_PER_W = B_SC // NW
CH = 200
NCHUNK = N // CH
LG = D // 16


def _table_body(ttm_ref, w_ref, b_ref, m_ref):
    m = lax.dot_general(ttm_ref[...], w_ref[...],
                        dimension_numbers=(((2,), (0,)), ((), ())),
                        preferred_element_type=jnp.float32)
    m_ref[...] = m + b_ref[...][None]


def _build_tables(ttm, w31, bias_row):
    tb = 8
    return pl.pallas_call(
        _table_body,
        grid=(B // tb,),
        in_specs=[
            pl.BlockSpec((tb, H, H), lambda i: (i, 0, 0)),
            pl.BlockSpec((H, D), lambda i: (0, 0)),
            pl.BlockSpec((1, D), lambda i: (0, 0)),
        ],
        out_specs=pl.BlockSpec((tb, H, D), lambda i: (i, 0, 0)),
        out_shape=jax.ShapeDtypeStruct((B, H, D), jnp.float32),
    )(ttm, w31, bias_row)


def _tc2_body(o_in_ref, m_ref, ix_ref, dem_ref, tw_ref, wx_ref, o_ref):
    del o_in_ref
    for j in range(_TB2):
        oh = (ix_ref[j][:, None]
              == lax.broadcasted_iota(jnp.int32, (N, H), 1)).astype(jnp.float32)
        base = jnp.dot(oh, m_ref[j], preferred_element_type=jnp.float32)
        tw = tw_ref[j]
        r = (base + dem_ref[j][:, None] * wx_ref[0][None]
             + tw[:, 0][:, None] * wx_ref[1][None]
             + tw[:, 1][:, None] * wx_ref[2][None])
        o_ref[j] = r


_TB2 = 8


def _tc2_fill(sc_out, tables, idx, dem, tw, wx3):
    g0 = B_SC // _TB2
    return pl.pallas_call(
        _tc2_body,
        grid=(B_TC // _TB2,),
        in_specs=[
            pl.BlockSpec(memory_space=pltpu.MemorySpace.HBM),
            pl.BlockSpec((_TB2, H, D), lambda i: (g0 + i, 0, 0)),
            pl.BlockSpec((_TB2, N), lambda i: (g0 + i, 0)),
            pl.BlockSpec((_TB2, N), lambda i: (g0 + i, 0)),
            pl.BlockSpec((_TB2, N, 2), lambda i: (g0 + i, 0, 0)),
            pl.BlockSpec((3, D), lambda i: (0, 0)),
        ],
        out_specs=pl.BlockSpec((_TB2, N, D), lambda i: (g0 + i, 0, 0)),
        out_shape=jax.ShapeDtypeStruct((B, N, D), jnp.float32),
        input_output_aliases={0: 0},
    )(sc_out, tables, idx, dem, tw, wx3)


def _sc_body(m_hbm, ix_hbm, pk_hbm, wx_hbm, out_hbm,
             table_v, in0, in1, ix0, ix1, out0, out1, wx_v,
             sin0, sin1, sout0, sout1):
    wid = lax.axis_index("s") * 2 + lax.axis_index("c")
    inb = (in0, in1)
    ixb = (ix0, ix1)
    outb = (out0, out1)
    sin = (sin0, sin1)
    sout = (sout0, sout1)

    pltpu.sync_copy(wx_hbm, wx_v)
    w31c = [wx_v[pl.ds(16 * l, 16)] for l in range(LG)]
    w32c = [wx_v[pl.ds(D + 16 * l, 16)] for l in range(LG)]
    w33c = [wx_v[pl.ds(2 * D + 16 * l, 16)] for l in range(LG)]
    cols = [lax.iota(jnp.int32, 16) + 16 * l for l in range(LG)]
    col1 = jnp.full((16,), 1, dtype=jnp.int32)
    col2 = jnp.full((16,), 2, dtype=jnp.int32)
    col3 = jnp.full((16,), 3, dtype=jnp.int32)
    zeros = jnp.zeros((16,), dtype=jnp.int32)

    b0 = wid * NB_PER_W

    def start_in(b, ci, p):
        base = b * N + ci * CH
        pltpu.async_copy(pk_hbm.at[pl.ds(base, CH)], inb[p], sin[p])
        pltpu.async_copy(ix_hbm.at[pl.ds(base, CH)], ixb[p], sin[p])

    start_in(b0, 0, 0)

    def batch_body(bi, _):
        for sub in range(2):
            b = b0 + bi * 2 + sub
            pltpu.sync_copy(m_hbm.at[b], table_v)

            for ci in range(NCHUNK):
                q = sub * NCHUNK + ci
                p = q % 2
                base = b * N + ci * CH
                if ci + 1 < NCHUNK:
                    start_in(b, ci + 1, (q + 1) % 2)
                else:
                    @pl.when(bi * 2 + sub + 1 < NB_PER_W)
                    def _():
                        start_in(b + 1, 0, (q + 1) % 2)

                pltpu.make_async_copy(
                    pk_hbm.at[pl.ds(base, CH)], inb[p], sin[p]).wait()
                pltpu.make_async_copy(
                    ix_hbm.at[pl.ds(base, CH)], ixb[p], sin[p]).wait()
                if q >= 2:
                    pltpu.make_async_copy(
                        outb[p], out_hbm.at[pl.ds(base * D, CH * D)],
                        sout[p]).wait()
                else:
                    @pl.when(bi > 0)
                    def _():
                        pltpu.make_async_copy(
                            outb[p], out_hbm.at[pl.ds(base * D, CH * D)],
                            sout[p]).wait()

                def node_body(n, sn):
                    row = plsc.load_gather(ixb[p], [sn])
                    dem = plsc.load_gather(inb[p], [sn, zeros])
                    t0 = plsc.load_gather(inb[p], [sn, col1])
                    t1 = plsc.load_gather(inb[p], [sn, col2])
                    gs = [plsc.load_gather(table_v, [row | cols[l]])
                          for l in range(LG)]
                    ms = [(dem * w31c[l] + t0 * w32c[l]) + t1 * w33c[l]
                          for l in range(LG)]
                    for l in range(LG):
                        outb[p][pl.ds(n * D + 16 * l, 16)] = gs[l] + ms[l]
                    return sn + 1

                lax.fori_loop(0, CH, node_body, zeros, unroll=8)
                pltpu.async_copy(
                    outb[p], out_hbm.at[pl.ds(base * D, CH * D)], sout[p])
        return 0

    lax.fori_loop(0, NB_PER_W // 2, batch_body, 0)

    tail = (b0 + NB_PER_W - 1) * N
    pltpu.make_async_copy(
        outb[0], out_hbm.at[pl.ds((tail + 4 * CH) * D, CH * D)], sout[0]).wait()
    pltpu.make_async_copy(
        outb[1], out_hbm.at[pl.ds((tail + 3 * CH) * D, CH * D)], sout[1]).wait()


_sc_call = functools.partial(
    pl.kernel,
    out_type=jax.ShapeDtypeStruct((B * N * D,), jnp.float32),
    mesh=plsc.VectorSubcoreMesh(core_axis_name="c", subcore_axis_name="s"),
    compiler_params=pltpu.CompilerParams(needs_layout_passes=False),
    scratch_types=[
        pltpu.VMEM((H * D,), jnp.float32),
        pltpu.VMEM((CH, 4), jnp.float32),
        pltpu.VMEM((CH, 4), jnp.float32),
        pltpu.VMEM((CH,), jnp.int32),
        pltpu.VMEM((CH,), jnp.int32),
        pltpu.VMEM((CH * D,), jnp.float32),
        pltpu.VMEM((CH * D,), jnp.float32),
        pltpu.VMEM((3 * D,), jnp.float32),
        pltpu.SemaphoreType.DMA,
        pltpu.SemaphoreType.DMA,
        pltpu.SemaphoreType.DMA,
        pltpu.SemaphoreType.DMA,
    ],
)(_sc_body)


@jax.jit
def kernel(travel_time_matrix, h3_indices, demand, time_windows, W, b):
    tables = _build_tables(travel_time_matrix, W[:H], b[None])
    idxw = (h3_indices.astype(jnp.int32) << 7).reshape(B * N)
    packed = jnp.concatenate(
        [demand[..., None], time_windows,
         jnp.zeros((B, N, 1), jnp.float32)], axis=-1).reshape(B * N, 4)
    wx = jnp.concatenate([W[H], W[H + 1], W[H + 2]])
    out = _sc_call(tables.reshape(B, H * D), idxw, packed, wx)
    out = _tc2_fill(out.reshape(B, N, D), tables,
                    h3_indices.astype(jnp.int32), demand, time_windows,
                    W[H:H + 3])
    return out
